# baseline (speedup 1.0000x reference)
import jax
import jax.numpy as jnp
from jax.experimental import pallas as pl
from jax.experimental.pallas import tpu as pltpu

_ROWS = 2 * 8192
_COLS = 4096
_CHUNK_ROWS = 2040
_NBUF = 2
_CHUNKS = []
_off = 0
while _off < _ROWS:
    _CHUNKS.append((_off, min(_CHUNK_ROWS, _ROWS - _off)))
    _off += _CHUNK_ROWS
_NCH = len(_CHUNKS)


def _ring_body(i_ref, o_ref, bufs, lsem, ssem):
    def ld(i, slot):
        off, rows = _CHUNKS[i]
        return pltpu.make_async_copy(
            i_ref.at[pl.ds(off, rows)], bufs.at[slot, pl.ds(0, rows)], lsem.at[slot]
        )

    def st(i, slot):
        off, rows = _CHUNKS[i]
        return pltpu.make_async_copy(
            bufs.at[slot, pl.ds(0, rows)], o_ref.at[pl.ds(off, rows)], ssem.at[slot]
        )

    for b in range(_NBUF):
        ld(b, b).start()

    for i in range(_NCH):
        slot = i % _NBUF
        ld(i, slot).wait()
        st(i, slot).start()
        nxt = i + _NBUF
        if nxt < _NCH:
            st(i, slot).wait()
            ld(nxt, slot).start()

    for i in range(max(0, _NCH - _NBUF), _NCH):
        st(i, i % _NBUF).wait()


def kernel(x, bit, alpha):
    del bit, alpha
    x2 = x.reshape(_ROWS, _COLS)
    out = pl.pallas_call(
        _ring_body,
        in_specs=[pl.BlockSpec(memory_space=pl.ANY)],
        out_specs=pl.BlockSpec(memory_space=pl.ANY),
        out_shape=jax.ShapeDtypeStruct((_ROWS, _COLS), x.dtype),
        scratch_shapes=[
            pltpu.VMEM((_NBUF, _CHUNK_ROWS, _COLS), jnp.float32),
            pltpu.SemaphoreType.DMA((_NBUF,)),
            pltpu.SemaphoreType.DMA((_NBUF,)),
        ],
        compiler_params=pltpu.CompilerParams(vmem_limit_bytes=100 * 1024 * 1024),
    )(x2)
    return out.reshape(x.shape)


# speedup vs baseline: 1.0108x; 1.0108x over previous
import jax
import jax.numpy as jnp
from jax.experimental import pallas as pl
from jax.experimental.pallas import tpu as pltpu

_ROWS = 2 * 8192
_COLS = 4096
_CHUNK_ROWS = 1360
_NBUF = 3
_CHUNKS = []
_off = 0
while _off < _ROWS:
    _CHUNKS.append((_off, min(_CHUNK_ROWS, _ROWS - _off)))
    _off += _CHUNK_ROWS
_NCH = len(_CHUNKS)


def _ring_body(i_ref, o_ref, bufs, lsem, ssem):
    def ld(i, slot):
        off, rows = _CHUNKS[i]
        return pltpu.make_async_copy(
            i_ref.at[pl.ds(off, rows)], bufs.at[slot, pl.ds(0, rows)], lsem.at[slot]
        )

    def st(i, slot):
        off, rows = _CHUNKS[i]
        return pltpu.make_async_copy(
            bufs.at[slot, pl.ds(0, rows)], o_ref.at[pl.ds(off, rows)], ssem.at[slot]
        )

    for b in range(_NBUF):
        ld(b, b).start()

    for i in range(_NCH):
        slot = i % _NBUF
        ld(i, slot).wait()
        st(i, slot).start()
        nxt = i + _NBUF
        if nxt < _NCH:
            st(i, slot).wait()
            ld(nxt, slot).start()

    for i in range(max(0, _NCH - _NBUF), _NCH):
        st(i, i % _NBUF).wait()


def kernel(x, bit, alpha):
    del bit, alpha
    x2 = x.reshape(_ROWS, _COLS)
    out = pl.pallas_call(
        _ring_body,
        in_specs=[pl.BlockSpec(memory_space=pl.ANY)],
        out_specs=pl.BlockSpec(memory_space=pl.ANY),
        out_shape=jax.ShapeDtypeStruct((_ROWS, _COLS), x.dtype),
        scratch_shapes=[
            pltpu.VMEM((_NBUF, _CHUNK_ROWS, _COLS), jnp.float32),
            pltpu.SemaphoreType.DMA((_NBUF,)),
            pltpu.SemaphoreType.DMA((_NBUF,)),
        ],
        compiler_params=pltpu.CompilerParams(vmem_limit_bytes=100 * 1024 * 1024),
    )(x2)
    return out.reshape(x.shape)
